# baseline (speedup 1.0000x reference)
import functools

import jax
import jax.numpy as jnp
from jax import lax
from jax.experimental import pallas as pl
from jax.experimental.pallas import tpu as pltpu
from jax.experimental.pallas import tpu_sc as plsc

H = 50
WPR = 56
NBUF = 8


def _coef_body(hist_ref, exits_ref, out_ref):
    ht = hist_ref[...]
    nh = ht.shape[0]
    cur = ht[nh - 1]
    ise = ht == exits_ref[0]
    for e in range(1, 16):
        ise = ise | (ht == exits_ref[e])
    slots = []
    for j in range(nh):
        hj = ht[j]
        if j == nh - 1:
            c = 0.5 - ise[j].astype(jnp.float32)
        else:
            dup = jnp.zeros(hj.shape, jnp.bool_) if j == 0 else jnp.any(
                ht[:j] == hj[None], axis=0
            )
            keep = ~dup & (hj != cur)
            c = jnp.where(keep, 0.1 - ise[j].astype(jnp.float32), 0.0)
        slots.append(c)
    for _ in range(WPR - nh):
        slots.append(jnp.zeros(cur.shape, jnp.float32))
    out_ref[...] = jnp.stack(slots, axis=0)


def _coefficients(hist, exits):
    B, nh = hist.shape
    hist_t = hist.T.reshape(nh, B // 128, 128)
    coef_t = pl.pallas_call(
        _coef_body,
        in_specs=[
            pl.BlockSpec(memory_space=pltpu.VMEM),
            pl.BlockSpec(memory_space=pltpu.SMEM),
        ],
        out_shape=jax.ShapeDtypeStruct((WPR, B // 128, 128), jnp.float32),
    )(hist_t, exits)
    return coef_t.reshape(WPR, B).T


def _bag(W1, exits, idx_flat, coef_flat, B):
    D = W1.shape[1]
    NE = exits.shape[0]
    DC = D // 16
    info = plsc.get_sparse_core_info()
    NC, NS = info.num_cores, info.num_subcores
    NW = NC * NS
    RPW = B // NW
    GROUPS = RPW // NBUF
    mesh = plsc.VectorSubcoreMesh(core_axis_name="c", subcore_axis_name="s")

    @functools.partial(
        pl.kernel,
        out_type=jax.ShapeDtypeStruct((B, D), jnp.float32),
        mesh=mesh,
        scratch_types=[
            pltpu.VMEM((RPW * WPR,), jnp.int32),
            pltpu.VMEM((RPW * WPR + 16,), jnp.float32),
            pltpu.VMEM((NE,), jnp.int32),
            pltpu.VMEM((NE, D), jnp.float32),
            [pltpu.VMEM((H, D), jnp.float32) for _ in range(NBUF)],
            pltpu.VMEM((RPW, D), jnp.float32),
            [pltpu.SemaphoreType.DMA for _ in range(NBUF)],
            pltpu.SemaphoreType.DMA,
        ],
    )
    def k(w1_hbm, ex_hbm, idx_hbm, coef_hbm, out_hbm,
          idx_v, coef_v, ex_v, ebuf, bufs, out_v, sems, esem):
        wid = lax.axis_index("s") * NC + lax.axis_index("c")
        base_e = wid * RPW * WPR

        pltpu.sync_copy(idx_hbm.at[pl.ds(base_e, RPW * WPR)], idx_v)
        pltpu.sync_copy(
            coef_hbm.at[pl.ds(base_e, RPW * WPR)],
            coef_v.at[pl.ds(0, RPW * WPR)],
        )
        pltpu.sync_copy(ex_hbm, ex_v)
        pltpu.async_copy(w1_hbm.at[ex_v], ebuf, esem).wait()

        e_acc = []
        for kk in range(DC):
            s = ebuf[0, pl.ds(kk * 16, 16)]
            for e in range(1, NE):
                s = s + ebuf[e, pl.ds(kk * 16, 16)]
            e_acc.append(s)

        def fire(row, slot):
            pltpu.async_copy(
                w1_hbm.at[idx_v.at[pl.ds(row * WPR, H)]], bufs[slot], sems[slot]
            )

        def drain(row, slot):
            pltpu.make_async_copy(
                w1_hbm.at[idx_v.at[pl.ds(row * WPR, H)]], bufs[slot], sems[slot]
            ).wait()

        for b in range(NBUF):
            fire(b, b)

        def outer(g, e):
            for b in range(NBUF):
                row = g * NBUF + b
                drain(row, b)
                buf = bufs[b]

                def chunk(cc, accs, _buf=buf, _row=row):
                    cw = coef_v[pl.ds(_row * WPR + cc * 10, 16)]
                    accs = list(accs)
                    for jj in range(10):
                        c = cw[jj]
                        for kk in range(DC):
                            accs[kk] = accs[kk] + c * _buf[
                                cc * 10 + jj, pl.ds(kk * 16, 16)
                            ]
                    return tuple(accs)

                accs = lax.fori_loop(0, H // 10, chunk, tuple(e))
                for kk in range(DC):
                    out_v[row, pl.ds(kk * 16, 16)] = accs[kk]

                @pl.when(row + NBUF < RPW)
                def _():
                    fire(row + NBUF, b)

            return e

        lax.fori_loop(0, GROUPS, outer, tuple(e_acc))
        pltpu.sync_copy(out_v, out_hbm.at[pl.ds(wid * RPW, RPW)])

    return k(W1, exits, idx_flat, coef_flat)


def _mlp_body(s_ref, b1_ref, w2_ref, b2_ref, out_ref):
    h = jnp.maximum(s_ref[...] + b1_ref[...], 0.0)
    o = lax.dot_general(
        h, w2_ref[...], (((1,), (0,)), ((), ())), preferred_element_type=jnp.float32
    )
    out_ref[...] = jnp.maximum(o + b2_ref[...], 0.0)


def _mlp(S, b1, W2, b2):
    B, D = S.shape
    O = W2.shape[1]
    return pl.pallas_call(
        _mlp_body,
        out_shape=jax.ShapeDtypeStruct((B, O), jnp.float32),
    )(S, b1.reshape(1, D), W2, b2.reshape(1, O))


def kernel(attacker_history, exits, W1, b1, W2, b2):
    hist = attacker_history.astype(jnp.int32)
    ex = exits.astype(jnp.int32)
    B, nh = hist.shape
    idx = jnp.concatenate([hist, jnp.zeros((B, WPR - nh), jnp.int32)], axis=1)
    coef = _coefficients(hist, ex)
    S = _bag(W1, ex, idx.reshape(-1), coef.reshape(-1), B)
    return _mlp(S, b1, W2, b2)


# speedup vs baseline: 1.0422x; 1.0422x over previous
import functools

import jax
import jax.numpy as jnp
from jax import lax
from jax.experimental import pallas as pl
from jax.experimental.pallas import tpu as pltpu
from jax.experimental.pallas import tpu_sc as plsc

H = 50
WPR = 56
NBUF = 4


def _coef_body(hist_ref, exits_ref, out_ref):
    ht = hist_ref[...]
    nh = ht.shape[0]
    cur = ht[nh - 1]
    ise = ht == exits_ref[0]
    for e in range(1, 16):
        ise = ise | (ht == exits_ref[e])
    slots = []
    for j in range(nh):
        hj = ht[j]
        if j == nh - 1:
            c = 0.5 - ise[j].astype(jnp.float32)
        else:
            dup = jnp.zeros(hj.shape, jnp.bool_) if j == 0 else jnp.any(
                ht[:j] == hj[None], axis=0
            )
            keep = ~dup & (hj != cur)
            c = jnp.where(keep, 0.1 - ise[j].astype(jnp.float32), 0.0)
        slots.append(c)
    for _ in range(WPR - nh):
        slots.append(jnp.zeros(cur.shape, jnp.float32))
    out_ref[...] = jnp.stack(slots, axis=0)


def _coefficients(hist, exits):
    B, nh = hist.shape
    hist_t = hist.T.reshape(nh, B // 128, 128)
    coef_t = pl.pallas_call(
        _coef_body,
        in_specs=[
            pl.BlockSpec(memory_space=pltpu.VMEM),
            pl.BlockSpec(memory_space=pltpu.SMEM),
        ],
        out_shape=jax.ShapeDtypeStruct((WPR, B // 128, 128), jnp.float32),
    )(hist_t, exits)
    return coef_t.reshape(WPR, B).T


def _bag(W1, exits, idx_flat, coef_flat, B):
    D = W1.shape[1]
    NE = exits.shape[0]
    DC = D // 16
    info = plsc.get_sparse_core_info()
    NC, NS = info.num_cores, info.num_subcores
    NW = NC * NS
    RPW = B // NW
    GROUPS = RPW // NBUF
    mesh = plsc.VectorSubcoreMesh(core_axis_name="c", subcore_axis_name="s")

    @functools.partial(
        pl.kernel,
        out_type=jax.ShapeDtypeStruct((B, D), jnp.float32),
        mesh=mesh,
        scratch_types=[
            pltpu.VMEM((RPW * WPR,), jnp.int32),
            pltpu.VMEM((RPW * WPR + 16,), jnp.float32),
            pltpu.VMEM((NE,), jnp.int32),
            pltpu.VMEM((NE, D), jnp.float32),
            [pltpu.VMEM((H, D), jnp.float32) for _ in range(NBUF)],
            pltpu.VMEM((RPW, D), jnp.float32),
            [pltpu.SemaphoreType.DMA for _ in range(NBUF)],
            pltpu.SemaphoreType.DMA,
        ],
    )
    def k(w1_hbm, ex_hbm, idx_hbm, coef_hbm, out_hbm,
          idx_v, coef_v, ex_v, ebuf, bufs, out_v, sems, esem):
        wid = lax.axis_index("s") * NC + lax.axis_index("c")
        base_e = wid * RPW * WPR

        pltpu.sync_copy(idx_hbm.at[pl.ds(base_e, RPW * WPR)], idx_v)
        pltpu.sync_copy(
            coef_hbm.at[pl.ds(base_e, RPW * WPR)],
            coef_v.at[pl.ds(0, RPW * WPR)],
        )
        pltpu.sync_copy(ex_hbm, ex_v)
        pltpu.async_copy(w1_hbm.at[ex_v], ebuf, esem).wait()

        e_acc = []
        for kk in range(DC):
            s = ebuf[0, pl.ds(kk * 16, 16)]
            for e in range(1, NE):
                s = s + ebuf[e, pl.ds(kk * 16, 16)]
            e_acc.append(s)

        def fire(row, slot):
            pltpu.async_copy(
                w1_hbm.at[idx_v.at[pl.ds(row * WPR, H)]], bufs[slot], sems[slot]
            )

        def drain(row, slot):
            pltpu.make_async_copy(
                w1_hbm.at[idx_v.at[pl.ds(row * WPR, H)]], bufs[slot], sems[slot]
            ).wait()

        for b in range(NBUF):
            fire(b, b)

        def outer(g, e):
            for b in range(NBUF):
                row = g * NBUF + b
                drain(row, b)
                buf = bufs[b]

                def chunk(cc, accs, _buf=buf, _row=row):
                    cw0 = coef_v[pl.ds(_row * WPR + cc * 25, 16)]
                    cw1 = coef_v[pl.ds(_row * WPR + cc * 25 + 16, 16)]
                    accs = list(accs)
                    for jj in range(25):
                        c = cw0[jj] if jj < 16 else cw1[jj - 16]
                        for kk in range(DC):
                            accs[kk] = accs[kk] + c * _buf[
                                cc * 25 + jj, pl.ds(kk * 16, 16)
                            ]
                    return tuple(accs)

                accs = lax.fori_loop(0, H // 25, chunk, tuple(e))
                for kk in range(DC):
                    out_v[row, pl.ds(kk * 16, 16)] = accs[kk]

                @pl.when(row + NBUF < RPW)
                def _():
                    fire(row + NBUF, b)

            return e

        lax.fori_loop(0, GROUPS, outer, tuple(e_acc))
        pltpu.sync_copy(out_v, out_hbm.at[pl.ds(wid * RPW, RPW)])

    return k(W1, exits, idx_flat, coef_flat)


def _mlp_body(s_ref, b1_ref, w2_ref, b2_ref, out_ref):
    h = jnp.maximum(s_ref[...] + b1_ref[...], 0.0)
    o = lax.dot_general(
        h, w2_ref[...], (((1,), (0,)), ((), ())), preferred_element_type=jnp.float32
    )
    out_ref[...] = jnp.maximum(o + b2_ref[...], 0.0)


def _mlp(S, b1, W2, b2):
    B, D = S.shape
    O = W2.shape[1]
    return pl.pallas_call(
        _mlp_body,
        out_shape=jax.ShapeDtypeStruct((B, O), jnp.float32),
    )(S, b1.reshape(1, D), W2, b2.reshape(1, O))


def kernel(attacker_history, exits, W1, b1, W2, b2):
    hist = attacker_history.astype(jnp.int32)
    ex = exits.astype(jnp.int32)
    B, nh = hist.shape
    idx = jnp.concatenate([hist, jnp.zeros((B, WPR - nh), jnp.int32)], axis=1)
    coef = _coefficients(hist, ex)
    S = _bag(W1, ex, idx.reshape(-1), coef.reshape(-1), B)
    return _mlp(S, b1, W2, b2)


# speedup vs baseline: 1.0517x; 1.0092x over previous
import functools

import jax
import jax.numpy as jnp
from jax import lax
from jax.experimental import pallas as pl
from jax.experimental.pallas import tpu as pltpu
from jax.experimental.pallas import tpu_sc as plsc

H = 50
WPR = 56
NBUF = 4


def _coef_body(hist_ref, exits_ref, out_ref):
    ht = hist_ref[...]
    nh = ht.shape[0]
    cur = ht[nh - 1]
    ise = ht == exits_ref[0]
    for e in range(1, 16):
        ise = ise | (ht == exits_ref[e])
    slots = []
    for j in range(nh):
        hj = ht[j]
        if j == nh - 1:
            c = 0.5 - ise[j].astype(jnp.float32)
        else:
            dup = jnp.zeros(hj.shape, jnp.bool_) if j == 0 else jnp.any(
                ht[:j] == hj[None], axis=0
            )
            keep = ~dup & (hj != cur)
            c = jnp.where(keep, 0.1 - ise[j].astype(jnp.float32), 0.0)
        slots.append(c)
    for _ in range(WPR - nh):
        slots.append(jnp.zeros(cur.shape, jnp.float32))
    out_ref[...] = jnp.stack(slots, axis=0)


def _coefficients(hist, exits):
    B, nh = hist.shape
    hist_t = hist.T.reshape(nh, B // 128, 128)
    coef_t = pl.pallas_call(
        _coef_body,
        in_specs=[
            pl.BlockSpec(memory_space=pltpu.VMEM),
            pl.BlockSpec(memory_space=pltpu.SMEM),
        ],
        out_shape=jax.ShapeDtypeStruct((WPR, B // 128, 128), jnp.float32),
    )(hist_t, exits)
    return coef_t.reshape(WPR, B).T


def _bag(W1, exits, idx_flat, coef_flat, B):
    D = W1.shape[1]
    NE = exits.shape[0]
    DC = D // 16
    info = plsc.get_sparse_core_info()
    NC, NS = info.num_cores, info.num_subcores
    NW = NC * NS
    RPW = B // NW
    GROUPS = RPW // NBUF
    mesh = plsc.VectorSubcoreMesh(core_axis_name="c", subcore_axis_name="s")

    @functools.partial(
        pl.kernel,
        out_type=jax.ShapeDtypeStruct((B, D), jnp.float32),
        mesh=mesh,
        scratch_types=[
            pltpu.VMEM((RPW * WPR,), jnp.int32),
            pltpu.VMEM((RPW * WPR + 16,), jnp.float32),
            pltpu.VMEM((NE,), jnp.int32),
            pltpu.VMEM((NE, D), jnp.float32),
            [pltpu.VMEM((H, D), jnp.float32) for _ in range(NBUF)],
            pltpu.VMEM((RPW, D), jnp.float32),
            [pltpu.SemaphoreType.DMA for _ in range(NBUF)],
            pltpu.SemaphoreType.DMA,
        ],
    )
    def k(w1_hbm, ex_hbm, idx_hbm, coef_hbm, out_hbm,
          idx_v, coef_v, ex_v, ebuf, bufs, out_v, sems, esem):
        wid = lax.axis_index("s") * NC + lax.axis_index("c")
        base_e = wid * RPW * WPR

        pltpu.sync_copy(idx_hbm.at[pl.ds(base_e, RPW * WPR)], idx_v)
        pltpu.sync_copy(
            coef_hbm.at[pl.ds(base_e, RPW * WPR)],
            coef_v.at[pl.ds(0, RPW * WPR)],
        )
        pltpu.sync_copy(ex_hbm, ex_v)
        pltpu.async_copy(w1_hbm.at[ex_v], ebuf, esem).wait()

        e_acc = []
        for kk in range(DC):
            s = ebuf[0, pl.ds(kk * 16, 16)]
            for e in range(1, NE):
                s = s + ebuf[e, pl.ds(kk * 16, 16)]
            e_acc.append(s)

        def fire(row, slot):
            pltpu.async_copy(
                w1_hbm.at[idx_v.at[pl.ds(row * WPR, H)]], bufs[slot], sems[slot]
            )

        def drain(row, slot):
            pltpu.make_async_copy(
                w1_hbm.at[idx_v.at[pl.ds(row * WPR, H)]], bufs[slot], sems[slot]
            ).wait()

        for b in range(NBUF):
            fire(b, b)

        def outer(g, e):
            for b in range(NBUF):
                row = g * NBUF + b
                drain(row, b)
                buf = bufs[b]

                def chunk(cc, accs, _buf=buf, _row=row):
                    cw = coef_v[pl.ds(_row * WPR + cc * 10, 16)]
                    accs = list(accs)
                    for jj in range(10):
                        c = cw[jj]
                        for kk in range(DC):
                            accs[kk] = accs[kk] + c * _buf[
                                cc * 10 + jj, pl.ds(kk * 16, 16)
                            ]
                    return tuple(accs)

                accs = lax.fori_loop(0, H // 10, chunk, tuple(e))
                for kk in range(DC):
                    out_v[row, pl.ds(kk * 16, 16)] = accs[kk]

                @pl.when(row + NBUF < RPW)
                def _():
                    fire(row + NBUF, b)

            return e

        lax.fori_loop(0, GROUPS, outer, tuple(e_acc))
        pltpu.sync_copy(out_v, out_hbm.at[pl.ds(wid * RPW, RPW)])

    return k(W1, exits, idx_flat, coef_flat)


def _mlp_body(s_ref, b1_ref, w2_ref, b2_ref, out_ref):
    h = jnp.maximum(s_ref[...] + b1_ref[...], 0.0)
    o = lax.dot_general(
        h, w2_ref[...], (((1,), (0,)), ((), ())), preferred_element_type=jnp.float32
    )
    out_ref[...] = jnp.maximum(o + b2_ref[...], 0.0)


def _mlp(S, b1, W2, b2):
    B, D = S.shape
    O = W2.shape[1]
    return pl.pallas_call(
        _mlp_body,
        out_shape=jax.ShapeDtypeStruct((B, O), jnp.float32),
    )(S, b1.reshape(1, D), W2, b2.reshape(1, O))


def kernel(attacker_history, exits, W1, b1, W2, b2):
    hist = attacker_history.astype(jnp.int32)
    ex = exits.astype(jnp.int32)
    B, nh = hist.shape
    idx = jnp.concatenate([hist, jnp.zeros((B, WPR - nh), jnp.int32)], axis=1)
    coef = _coefficients(hist, ex)
    S = _bag(W1, ex, idx.reshape(-1), coef.reshape(-1), B)
    return _mlp(S, b1, W2, b2)
